# baseline (speedup 1.0000x reference)
import jax
import jax.numpy as jnp
from jax.experimental import pallas as pl
from jax.experimental.pallas import tpu as pltpu

_TILE = 4096


def _body(x_ref, ma_ref, cbp_ref, mb_ref, mc_ref, o_ref):
    t = jnp.dot(x_ref[...], ma_ref[...], preferred_element_type=jnp.float32)
    u = jnp.maximum(t + cbp_ref[...], 0.0)
    w = jnp.dot(u, mb_ref[...], preferred_element_type=jnp.float32)
    z = jnp.maximum(w, 0.0)
    o = jnp.dot(z, mc_ref[...], preferred_element_type=jnp.float32)
    o_ref[...] = jnp.maximum(o, 0.0)


def kernel(x, conv_w, conv_b, W0, b0, W1, b1, W2, b2):
    B, I = x.shape
    D = W2.shape[0]
    f32 = jnp.float32
    icols = jnp.arange(16) * 8
    k16 = jnp.arange(16)
    ma = jnp.zeros((I, 32), f32).at[icols, k16].set(conv_w[icols])
    cbp = jnp.zeros((1, 32), f32).at[0, :16].set(conv_b[icols]).at[0, 16].set(1.0)
    mb = (jnp.zeros((32, 32), f32)
          .at[:16, :16].set(W1[::16, :16].T)
          .at[16, :16].set(b1[::16])
          .at[16, 16].set(1.0))
    mc = jnp.zeros((32, D), f32).at[:16, :].set(W2[:, :16].T).at[16, :].set(b2)

    grid = (B // _TILE,)
    return pl.pallas_call(
        _body,
        grid=grid,
        in_specs=[
            pl.BlockSpec((_TILE, I), lambda i: (i, 0)),
            pl.BlockSpec((I, 32), lambda i: (0, 0)),
            pl.BlockSpec((1, 32), lambda i: (0, 0)),
            pl.BlockSpec((32, 32), lambda i: (0, 0)),
            pl.BlockSpec((32, D), lambda i: (0, 0)),
        ],
        out_specs=pl.BlockSpec((_TILE, D), lambda i: (i, 0)),
        out_shape=jax.ShapeDtypeStruct((B, D), x.dtype),
        compiler_params=pltpu.CompilerParams(
            dimension_semantics=("arbitrary",),
        ),
    )(x, ma, cbp, mb, mc)
